# Initial kernel scaffold; baseline (speedup 1.0000x reference)
import jax
import jax.numpy as jnp
from jax.experimental import pallas as pl


def kernel(x, emb_table, W, b):
    raise NotImplementedError("write your pallas kernel here")


# speedup vs baseline: 2.9556x; 2.9556x over previous
import functools

import jax
import jax.numpy as jnp
from jax import lax
from jax.experimental import pallas as pl
from jax.experimental.pallas import tpu as pltpu
from jax.experimental.pallas import tpu_sc as plsc

_NC = 2
_NS = 16
_NW = _NC * _NS
_LANES = 16

_B = 16384
_L = 50
_EMB = 32

_RW = _B // _NW
_G = 16
_NGRP = _RW // _G
_IPG = _G * _L
_CHUNKS = [(i * 128, 128) for i in range(6)] + [(768, 32)]


def _worker_id():
    return lax.axis_index("s") * _NC + lax.axis_index("c")


def _sc_body(x_hbm, wb_hbm, emb_hbm, out_hbm,
             idx_v, rows0, rows1, wb_v, macc_v, out_v, sem0, sem1):
    wid = _worker_id()
    idx_base = wid * (_RW * _L)

    pltpu.sync_copy(x_hbm.at[pl.ds(idx_base, _RW * _L)], idx_v)
    pltpu.sync_copy(wb_hbm, wb_v)
    bvec = wb_v[pl.ds(_EMB * _LANES, _LANES)]
    lane = lax.iota(jnp.int32, _LANES)

    def fire(g, buf, sem):
        handles = []
        for off, sz in _CHUNKS:
            src = emb_hbm.at[idx_v.at[pl.ds(g * _IPG + off, sz)]]
            handles.append(pltpu.async_copy(src, buf.at[pl.ds(off, sz)], sem))
        return handles

    def compute(g, buf):
        def row_body(r, carry):
            base = r * _L
            zero = jnp.zeros((_LANES,), jnp.float32)

            def jstep(t, accs):
                a0, a1, a2, a3 = accs
                p = base + t * 10
                for u in range(0, 10, 2):
                    a0 = a0 + buf[p + u, pl.ds(0, _LANES)]
                    a1 = a1 + buf[p + u, pl.ds(_LANES, _LANES)]
                    a2 = a2 + buf[p + u + 1, pl.ds(0, _LANES)]
                    a3 = a3 + buf[p + u + 1, pl.ds(_LANES, _LANES)]
                return (a0, a1, a2, a3)

            a0, a1, a2, a3 = lax.fori_loop(
                0, _L // 10, jstep, (zero, zero, zero, zero))
            macc_v[r, pl.ds(0, _LANES)] = a0 + a2
            macc_v[r, pl.ds(_LANES, _LANES)] = a1 + a3
            return carry

        lax.fori_loop(0, _G, row_body, 0)
        ovec = jnp.zeros((_LANES,), jnp.float32)
        for k in range(_EMB):
            colv = plsc.load_gather(
                macc_v, [lane, jnp.full((_LANES,), k, jnp.int32)])
            ovec = ovec + colv * wb_v[pl.ds(k * _LANES, _LANES)]
        out_v[pl.ds(g * _G, _G)] = ovec * jnp.float32(1.0 / _L) + bvec

    for h in fire(0, rows0, sem0):
        h.wait()
    def pipe(p, carry):
        g0 = 2 * p
        g1 = 2 * p + 1
        h1 = fire(g1, rows1, sem1)
        compute(g0, rows0)
        for h in h1:
            h.wait()
        g2 = jnp.minimum(g0 + 2, _NGRP - 1)
        h2 = fire(g2, rows0, sem0)
        compute(g1, rows1)
        for h in h2:
            h.wait()
        return carry

    lax.fori_loop(0, _NGRP // 2, pipe, 0)
    pltpu.sync_copy(out_v, out_hbm.at[pl.ds(wid * _RW, _RW)])


@functools.partial(jax.jit, static_argnums=())
def _sc_pool(x_flat, wb, emb_table):
    mesh = plsc.VectorSubcoreMesh(core_axis_name="c", subcore_axis_name="s")
    return pl.kernel(
        _sc_body,
        out_type=jax.ShapeDtypeStruct((_B,), jnp.float32),
        mesh=mesh,
        compiler_params=pltpu.CompilerParams(
            needs_layout_passes=False, use_tc_tiling_on_sc=False),
        scratch_types=[
            pltpu.VMEM((_RW * _L,), jnp.int32),
            pltpu.VMEM((_IPG, _EMB), jnp.float32),
            pltpu.VMEM((_IPG, _EMB), jnp.float32),
            pltpu.VMEM(((_EMB + 1) * _LANES,), jnp.float32),
            pltpu.VMEM((_G, _EMB), jnp.float32),
            pltpu.VMEM((_RW,), jnp.float32),
            pltpu.SemaphoreType.DMA,
            pltpu.SemaphoreType.DMA,
        ],
    )(x_flat, wb, emb_table)


def kernel(x, emb_table, W, b):
    B, L = x.shape
    assert (B, L) == (_B, _L) and emb_table.shape[1] == _EMB
    x_flat = x.reshape(B * L).astype(jnp.int32)
    wb = jnp.concatenate([
        jnp.broadcast_to(W.reshape(_EMB, 1).astype(jnp.float32),
                         (_EMB, _LANES)).reshape(_EMB * _LANES),
        jnp.broadcast_to(b.astype(jnp.float32), (_LANES,)),
    ])
    return _sc_pool(x_flat, wb, emb_table)


# speedup vs baseline: 10.3988x; 3.5184x over previous
import functools

import jax
import jax.numpy as jnp
from jax import lax
from jax.experimental import pallas as pl
from jax.experimental.pallas import tpu as pltpu
from jax.experimental.pallas import tpu_sc as plsc

_NC = 2
_NS = 16
_NW = _NC * _NS
_LANES = 16

_B = 16384
_L = 50
_EMB = 32
_VOCAB = 1000000

_RW = _B // _NW
_G = 16
_NGRP = _RW // _G
_IPG = _G * _L
_CHUNKS = [(i * 128, 128) for i in range(6)] + [(768, 32)]

_BLK = 65536


def _tc_body(embT_ref, w_ref, t_ref):
    t_ref[...] = jnp.sum(embT_ref[...] * w_ref[...], axis=0)


@jax.jit
def _tc_matvec(emb_t, w):
    grid = (_VOCAB + _BLK - 1) // _BLK
    return pl.pallas_call(
        _tc_body,
        grid=(grid,),
        in_specs=[
            pl.BlockSpec((_EMB, _BLK), lambda i: (0, i)),
            pl.BlockSpec((_EMB, 1), lambda i: (0, 0)),
        ],
        out_specs=pl.BlockSpec((_BLK,), lambda i: (i,)),
        out_shape=jax.ShapeDtypeStruct((_VOCAB,), jnp.float32),
    )(emb_t, w)


def _worker_id():
    return lax.axis_index("s") * _NC + lax.axis_index("c")


def _sc_body(xt_hbm, b_hbm, t_hbm, out_hbm,
             idx_v, buf0, buf1, b_v, out_v, sem0, sem1):
    wid = _worker_id()
    idx_base = wid * (_RW * _L)

    pltpu.sync_copy(xt_hbm.at[pl.ds(idx_base, _RW * _L)], idx_v)
    pltpu.sync_copy(b_hbm, b_v)
    bvec = b_v[...]

    def fire(g, buf, sem):
        handles = []
        for off, sz in _CHUNKS:
            src = t_hbm.at[idx_v.at[pl.ds(g * _IPG + off, sz)]]
            handles.append(pltpu.async_copy(src, buf.at[pl.ds(off, sz)], sem))
        return handles

    def compute(g, buf):
        acc = buf[pl.ds(0, _LANES)]
        for j in range(1, _L):
            acc = acc + buf[pl.ds(j * _LANES, _LANES)]
        out_v[pl.ds(g * _G, _G)] = acc * jnp.float32(1.0 / _L) + bvec

    for h in fire(0, buf0, sem0):
        h.wait()

    def pipe(p, carry):
        g0 = 2 * p
        g1 = 2 * p + 1
        h1 = fire(g1, buf1, sem1)
        compute(g0, buf0)
        for h in h1:
            h.wait()
        g2 = jnp.minimum(g0 + 2, _NGRP - 1)
        h2 = fire(g2, buf0, sem0)
        compute(g1, buf1)
        for h in h2:
            h.wait()
        return carry

    lax.fori_loop(0, _NGRP // 2, pipe, 0)
    pltpu.sync_copy(out_v, out_hbm.at[pl.ds(wid * _RW, _RW)])


@jax.jit
def _sc_pool(xt_flat, b16, t):
    mesh = plsc.VectorSubcoreMesh(core_axis_name="c", subcore_axis_name="s")
    return pl.kernel(
        _sc_body,
        out_type=jax.ShapeDtypeStruct((_B,), jnp.float32),
        mesh=mesh,
        compiler_params=pltpu.CompilerParams(
            needs_layout_passes=False, use_tc_tiling_on_sc=False),
        scratch_types=[
            pltpu.VMEM((_RW * _L,), jnp.int32),
            pltpu.VMEM((_IPG,), jnp.float32),
            pltpu.VMEM((_IPG,), jnp.float32),
            pltpu.VMEM((_LANES,), jnp.float32),
            pltpu.VMEM((_RW,), jnp.float32),
            pltpu.SemaphoreType.DMA,
            pltpu.SemaphoreType.DMA,
        ],
    )(xt_flat, b16, t)


def kernel(x, emb_table, W, b):
    B, L = x.shape
    assert (B, L) == (_B, _L) and emb_table.shape == (_VOCAB, _EMB)
    t = _tc_matvec(emb_table.T, W.astype(jnp.float32))
    xt = (x.astype(jnp.int32)
          .reshape(_B // _G, _G, _L)
          .transpose(0, 2, 1)
          .reshape(_B * _L))
    b16 = jnp.broadcast_to(b.astype(jnp.float32), (_LANES,))
    return _sc_pool(xt, b16, t)


# speedup vs baseline: 12.2629x; 1.1793x over previous
import functools

import jax
import jax.numpy as jnp
from jax import lax
from jax.experimental import pallas as pl
from jax.experimental.pallas import tpu as pltpu
from jax.experimental.pallas import tpu_sc as plsc

_NC = 2
_NS = 16
_NW = _NC * _NS
_LANES = 16

_B = 16384
_L = 50
_EMB = 32
_VOCAB = 1000000

_RW = _B // _NW
_G = 16
_NGRP = _RW // _G
_IPG = _G * _L
_CHUNKS = [(i * 128, 128) for i in range(6)] + [(768, 32)]

_BLK = 65536


def _tc_body(embT_ref, w_ref, t_ref):
    t_ref[...] = jnp.sum(embT_ref[...] * w_ref[...], axis=0)


@jax.jit
def _tc_matvec(emb_t, w):
    grid = (_VOCAB + _BLK - 1) // _BLK
    return pl.pallas_call(
        _tc_body,
        grid=(grid,),
        in_specs=[
            pl.BlockSpec((_EMB, _BLK), lambda i: (0, i)),
            pl.BlockSpec((_EMB, 1), lambda i: (0, 0)),
        ],
        out_specs=pl.BlockSpec((_BLK,), lambda i: (i,)),
        out_shape=jax.ShapeDtypeStruct((_VOCAB,), jnp.float32),
    )(emb_t, w)


def _worker_id():
    return lax.axis_index("s") * _NC + lax.axis_index("c")


def _sc_body(xt_hbm, b_hbm, t_hbm, out_hbm,
             idx_v, buf0, buf1, b_v, out_v, sem0, sem1):
    wid = _worker_id()
    idx_base = wid * (_RW * _L)

    pltpu.sync_copy(xt_hbm.at[pl.ds(idx_base, _RW * _L)], idx_v)
    pltpu.sync_copy(b_hbm, b_v)
    bvec = b_v[...]

    def fire(g, buf, sem):
        handles = []
        for off, sz in _CHUNKS:
            src = t_hbm.at[idx_v.at[pl.ds(g * _IPG + off, sz)]]
            handles.append(pltpu.async_copy(src, buf.at[pl.ds(off, sz)], sem))
        return handles

    rowoff = lax.iota(jnp.int32, _LANES) * _L

    def compute(g, buf):
        acc = plsc.load_gather(buf, [rowoff])
        for j in range(1, _L):
            acc = acc + plsc.load_gather(buf, [rowoff + j])
        out_v[pl.ds(g * _G, _G)] = acc * jnp.float32(1.0 / _L) + bvec

    for h in fire(0, buf0, sem0):
        h.wait()

    def pipe(p, carry):
        g0 = 2 * p
        g1 = 2 * p + 1
        h1 = fire(g1, buf1, sem1)
        compute(g0, buf0)
        for h in h1:
            h.wait()
        g2 = jnp.minimum(g0 + 2, _NGRP - 1)
        h2 = fire(g2, buf0, sem0)
        compute(g1, buf1)
        for h in h2:
            h.wait()
        return carry

    lax.fori_loop(0, _NGRP // 2, pipe, 0)
    pltpu.sync_copy(out_v, out_hbm.at[pl.ds(wid * _RW, _RW)])


@jax.jit
def _sc_pool(xt_flat, b16, t):
    mesh = plsc.VectorSubcoreMesh(core_axis_name="c", subcore_axis_name="s")
    return pl.kernel(
        _sc_body,
        out_type=jax.ShapeDtypeStruct((_B,), jnp.float32),
        mesh=mesh,
        compiler_params=pltpu.CompilerParams(
            needs_layout_passes=False, use_tc_tiling_on_sc=False),
        scratch_types=[
            pltpu.VMEM((_RW * _L,), jnp.int32),
            pltpu.VMEM((_IPG,), jnp.float32),
            pltpu.VMEM((_IPG,), jnp.float32),
            pltpu.VMEM((_LANES,), jnp.float32),
            pltpu.VMEM((_RW,), jnp.float32),
            pltpu.SemaphoreType.DMA,
            pltpu.SemaphoreType.DMA,
        ],
    )(xt_flat, b16, t)


def kernel(x, emb_table, W, b):
    B, L = x.shape
    assert (B, L) == (_B, _L) and emb_table.shape == (_VOCAB, _EMB)
    t = _tc_matvec(emb_table.T, W.astype(jnp.float32))
    xt = x.astype(jnp.int32).reshape(_B * _L)
    b16 = jnp.broadcast_to(b.astype(jnp.float32), (_LANES,))
    return _sc_pool(xt, b16, t)


# speedup vs baseline: 14.4410x; 1.1776x over previous
import functools

import jax
import jax.numpy as jnp
from jax import lax
from jax.experimental import pallas as pl
from jax.experimental.pallas import tpu as pltpu
from jax.experimental.pallas import tpu_sc as plsc

_NC = 2
_NS = 16
_NW = _NC * _NS
_LANES = 16

_B = 16384
_L = 50
_EMB = 32
_VOCAB = 1000000

_RW = _B // _NW
_G = 16
_NGRP = _RW // _G
_IPG = _G * _L
_CHUNKS = [(i * 128, 128) for i in range(6)] + [(768, 32)]

_BLK = 65536


def _tc_body(embT_ref, w_ref, t_ref):
    t_ref[...] = jnp.sum(embT_ref[...] * w_ref[...], axis=0)


@jax.jit
def _tc_matvec(emb_t, w):
    grid = (_VOCAB + _BLK - 1) // _BLK
    return pl.pallas_call(
        _tc_body,
        grid=(grid,),
        in_specs=[
            pl.BlockSpec((_EMB, _BLK), lambda i: (0, i)),
            pl.BlockSpec((_EMB, 1), lambda i: (0, 0)),
        ],
        out_specs=pl.BlockSpec((_BLK,), lambda i: (i,)),
        out_shape=jax.ShapeDtypeStruct((_VOCAB,), jnp.float32),
    )(emb_t, w)


def _worker_id():
    return lax.axis_index("s") * _NC + lax.axis_index("c")


_NPH = 4
_GPP = _NGRP // _NPH
_EPP = _GPP * _IPG


def _sc_body(xt_hbm, b_hbm, t_hbm, out_hbm,
             idx_v, buf, b_v, out_v, *sems):
    wid = _worker_id()
    idx_base = wid * (_RW * _L)

    pltpu.sync_copy(xt_hbm.at[pl.ds(idx_base, _RW * _L)], idx_v)
    pltpu.sync_copy(b_hbm, b_v)
    bvec = b_v[...]

    def fire_group(sem):
        def body(g, carry):
            for off, sz in _CHUNKS:
                src = t_hbm.at[idx_v.at[pl.ds(g * _IPG + off, sz)]]
                pltpu.async_copy(src, buf.at[pl.ds(g * _IPG + off, sz)], sem)
            return carry
        return body

    for p in range(_NPH):
        lax.fori_loop(p * _GPP, (p + 1) * _GPP, fire_group(sems[p]), 0)

    rowoff = lax.iota(jnp.int32, _LANES) * _L

    def compute(g, carry):
        base = g * _IPG
        acc = plsc.load_gather(buf, [base + rowoff])
        for j in range(1, _L):
            acc = acc + plsc.load_gather(buf, [base + rowoff + j])
        out_v[pl.ds(g * _G, _G)] = acc * jnp.float32(1.0 / _L) + bvec
        return carry

    for p in range(_NPH):
        pltpu.make_async_copy(
            t_hbm.at[pl.ds(0, _EPP)],
            buf.at[pl.ds(p * _EPP, _EPP)],
            sems[p]).wait()
        lax.fori_loop(p * _GPP, (p + 1) * _GPP, compute, 0)

    pltpu.sync_copy(out_v, out_hbm.at[pl.ds(wid * _RW, _RW)])


@jax.jit
def _sc_pool(xt_flat, b16, t):
    mesh = plsc.VectorSubcoreMesh(core_axis_name="c", subcore_axis_name="s")
    return pl.kernel(
        _sc_body,
        out_type=jax.ShapeDtypeStruct((_B,), jnp.float32),
        mesh=mesh,
        compiler_params=pltpu.CompilerParams(
            needs_layout_passes=False, use_tc_tiling_on_sc=False),
        scratch_types=[
            pltpu.VMEM((_RW * _L,), jnp.int32),
            pltpu.VMEM((_RW * _L,), jnp.float32),
            pltpu.VMEM((_LANES,), jnp.float32),
            pltpu.VMEM((_RW,), jnp.float32),
        ] + [pltpu.SemaphoreType.DMA] * _NPH,
    )(xt_flat, b16, t)


def kernel(x, emb_table, W, b):
    B, L = x.shape
    assert (B, L) == (_B, _L) and emb_table.shape == (_VOCAB, _EMB)
    t = _tc_matvec(emb_table.T, W.astype(jnp.float32))
    xt = x.astype(jnp.int32).reshape(_B * _L)
    b16 = jnp.broadcast_to(b.astype(jnp.float32), (_LANES,))
    return _sc_pool(xt, b16, t)


# speedup vs baseline: 14.7880x; 1.0240x over previous
import functools

import jax
import jax.numpy as jnp
from jax import lax
from jax.experimental import pallas as pl
from jax.experimental.pallas import tpu as pltpu
from jax.experimental.pallas import tpu_sc as plsc

_NC = 2
_NS = 16
_NW = _NC * _NS
_LANES = 16

_B = 16384
_L = 50
_EMB = 32
_VOCAB = 1000000

_RW = _B // _NW
_G = 16
_NGRP = _RW // _G
_IPG = _G * _L
_CHUNKS = [(i * 128, 128) for i in range(6)] + [(768, 32)]

_BLK = 65536


def _tc_body(embT_ref, w_ref, t_ref):
    t_ref[...] = jnp.dot(w_ref[...], embT_ref[...],
                         preferred_element_type=jnp.float32)[0]


@jax.jit
def _tc_matvec(emb_t, w):
    grid = (_VOCAB + _BLK - 1) // _BLK
    return pl.pallas_call(
        _tc_body,
        grid=(grid,),
        in_specs=[
            pl.BlockSpec((_EMB, _BLK), lambda i: (0, i)),
            pl.BlockSpec((1, _EMB), lambda i: (0, 0)),
        ],
        out_specs=pl.BlockSpec((_BLK,), lambda i: (i,)),
        out_shape=jax.ShapeDtypeStruct((_VOCAB,), jnp.float32),
    )(emb_t, w)


def _worker_id():
    return lax.axis_index("s") * _NC + lax.axis_index("c")


_NPH = 4
_GPP = _NGRP // _NPH
_EPP = _GPP * _IPG


def _sc_body(xt_hbm, b_hbm, t_hbm, out_hbm,
             idx_v, buf, b_v, out_v, *sems):
    wid = _worker_id()
    idx_base = wid * (_RW * _L)

    pltpu.sync_copy(xt_hbm.at[pl.ds(idx_base, _RW * _L)], idx_v)
    pltpu.sync_copy(b_hbm, b_v)
    bvec = b_v[...]

    def fire_group(sem):
        def body(g, carry):
            for off, sz in _CHUNKS:
                src = t_hbm.at[idx_v.at[pl.ds(g * _IPG + off, sz)]]
                pltpu.async_copy(src, buf.at[pl.ds(g * _IPG + off, sz)], sem)
            return carry
        return body

    for p in range(_NPH):
        lax.fori_loop(p * _GPP, (p + 1) * _GPP, fire_group(sems[p]), 0)

    rowoff = lax.iota(jnp.int32, _LANES) * _L

    def compute(g, carry):
        base = g * _IPG
        acc = plsc.load_gather(buf, [base + rowoff])
        for j in range(1, _L):
            acc = acc + plsc.load_gather(buf, [base + rowoff + j])
        out_v[pl.ds(g * _G, _G)] = acc * jnp.float32(1.0 / _L) + bvec
        return carry

    for p in range(_NPH):
        pltpu.make_async_copy(
            t_hbm.at[pl.ds(0, _EPP)],
            buf.at[pl.ds(p * _EPP, _EPP)],
            sems[p]).wait()
        lax.fori_loop(p * _GPP, (p + 1) * _GPP, compute, 0)

    pltpu.sync_copy(out_v, out_hbm.at[pl.ds(wid * _RW, _RW)])


@jax.jit
def _sc_pool(xt_flat, b16, t):
    mesh = plsc.VectorSubcoreMesh(core_axis_name="c", subcore_axis_name="s")
    return pl.kernel(
        _sc_body,
        out_type=jax.ShapeDtypeStruct((_B,), jnp.float32),
        mesh=mesh,
        compiler_params=pltpu.CompilerParams(
            needs_layout_passes=False, use_tc_tiling_on_sc=False),
        scratch_types=[
            pltpu.VMEM((_RW * _L,), jnp.int32),
            pltpu.VMEM((_RW * _L,), jnp.float32),
            pltpu.VMEM((_LANES,), jnp.float32),
            pltpu.VMEM((_RW,), jnp.float32),
        ] + [pltpu.SemaphoreType.DMA] * _NPH,
    )(xt_flat, b16, t)


def kernel(x, emb_table, W, b):
    B, L = x.shape
    assert (B, L) == (_B, _L) and emb_table.shape == (_VOCAB, _EMB)
    t = _tc_matvec(emb_table.T, W.astype(jnp.float32).reshape(1, _EMB))
    xt = x.astype(jnp.int32).reshape(_B * _L)
    b16 = jnp.broadcast_to(b.astype(jnp.float32), (_LANES,))
    return _sc_pool(xt, b16, t)
